# baseline (speedup 1.0000x reference)
import functools

import jax
import jax.numpy as jnp
from jax import lax
from jax.experimental import pallas as pl
from jax.experimental.pallas import tpu as pltpu
from jax.experimental.pallas import tpu_sc as plsc

SEQ = 8192
BATCH = 4
DIM = 768
RED = SEQ // 2
NCORES = 2
NSUB = 16
LANES = 16

NVR = SEQ // LANES
CHUNK = SEQ // LANES
RADIX = 256
NPASS = 4


_SBLK = 1024


def _scores_body(x_ref, w_ref, o_ref):
    wb = jnp.broadcast_to(
        w_ref[...].astype(jnp.bfloat16).reshape(DIM, 1), (DIM, 8))
    for b in range(BATCH):
        xb = x_ref[:, b, :].astype(jnp.bfloat16)
        acc = lax.dot_general(
            xb, wb, (((1,), (0,)), ((), ())),
            preferred_element_type=jnp.float32)
        o_ref[b, :] = acc[:, 0]


def _scores_tc(x, w2):
    return pl.pallas_call(
        _scores_body,
        grid=(SEQ // _SBLK,),
        in_specs=[
            pl.BlockSpec((_SBLK, BATCH, DIM), lambda i: (i, 0, 0)),
            pl.BlockSpec((1, DIM), lambda i: (0, 0)),
        ],
        out_specs=pl.BlockSpec((BATCH, _SBLK), lambda i: (0, i)),
        out_shape=jax.ShapeDtypeStruct((BATCH, SEQ), jnp.float32),
    )(x, w2)


def _sc_body(scores_hbm, x_hbm, idx_hbm, w_hbm, xds_hbm,
             score_v, key_a, pos_a, key_b, pos_b, hist, rank_v,
             dclip_v, kept_i, kept_w, ids_v, gbuf, sem):
    cidx = lax.axis_index("c")
    sidx = lax.axis_index("s")
    lane = lax.iota(jnp.int32, LANES)

    @pl.when(sidx < 2)
    def _sort_phase():
        b = 2 * cidx + sidx
        pltpu.sync_copy(scores_hbm.at[b], score_v)

        def _mk(i, c):
            s = score_v[pl.ds(i * LANES, LANES)]
            bits = lax.bitcast_convert_type(s, jnp.int32)
            asr = lax.shift_right_arithmetic(bits, 31)
            key = bits ^ (jnp.bitwise_not(asr) & jnp.int32(0x7FFFFFFF))
            key_a[pl.ds(i * LANES, LANES)] = key
            pos_a[pl.ds(i * LANES, LANES)] = i * LANES + lane
            return c
        lax.fori_loop(0, NVR, _mk, 0)

        for p in range(NPASS):
            src_k, src_p = (key_a, pos_a) if p % 2 == 0 else (key_b, pos_b)
            dst_k, dst_p = (key_b, pos_b) if p % 2 == 0 else (key_a, pos_a)
            shift = 8 * p

            def _hz(j, c):
                hist[pl.ds(j * LANES, LANES)] = jnp.zeros((LANES,), jnp.int32)
                return c
            lax.fori_loop(0, RADIX * LANES // LANES, _hz, 0)

            def _h1(t, c):
                pp = lane * CHUNK + t
                k = plsc.load_gather(src_k, [pp])
                dg = lax.shift_right_logical(k, shift) & 255
                plsc.addupdate_scatter(hist, [dg * LANES + lane],
                                       jnp.ones((LANES,), jnp.int32))
                return c
            lax.fori_loop(0, CHUNK, _h1, 0)

            def _sc(j, carry):
                v = hist[pl.ds(j * LANES, LANES)]
                cs = plsc.cumsum(v)
                hist[pl.ds(j * LANES, LANES)] = carry + cs - v
                return carry + jnp.sum(v)
            lax.fori_loop(0, RADIX, _sc, jnp.int32(0))

            def _p1(t, c):
                pp = lane * CHUNK + t
                k = plsc.load_gather(src_k, [pp])
                pv = plsc.load_gather(src_p, [pp])
                dg = lax.shift_right_logical(k, shift) & 255
                slot = dg * LANES + lane
                off = plsc.load_gather(hist, [slot])
                plsc.store_scatter(dst_k, [off], k)
                plsc.store_scatter(dst_p, [off], pv)
                plsc.store_scatter(hist, [slot], off + 1)
                return c
            lax.fori_loop(0, CHUNK, _p1, 0)

        def _rk(k, c):
            pv = pos_a[pl.ds(k * LANES, LANES)]
            plsc.store_scatter(rank_v, [pv], k * LANES + lane)
            return c
        lax.fori_loop(0, NVR, _rk, 0)

        def _dc(k, c):
            pv = pos_a[pl.ds(RED + k * LANES, LANES)]
            s = plsc.load_gather(score_v, [pv])
            dclip_v[pl.ds(k * LANES, LANES)] = jnp.clip(s, 0.0, 1.0)
            return c
        lax.fori_loop(0, RED // LANES, _dc, 0)

        def _cp(i, off):
            r = rank_v[pl.ds(i * LANES, LANES)]
            msk = r < RED
            s = score_v[pl.ds(i * LANES, LANES)]
            dval = plsc.load_gather(dclip_v, [r & (RED - 1)])
            w = jnp.clip(s, 0.0, 1.0) - dval
            mi = msk.astype(jnp.int32)
            posn = off + plsc.cumsum(mi) - mi
            plsc.store_scatter(kept_i, [posn], i * LANES + lane, mask=msk)
            plsc.store_scatter(kept_w, [posn], w, mask=msk)
            return off + jnp.sum(mi)
        lax.fori_loop(0, NVR, _cp, jnp.int32(0))

        pltpu.sync_copy(kept_i, idx_hbm.at[b])
        pltpu.sync_copy(kept_w, w_hbm.at[b])

    plsc.subcore_barrier()

    gb = 2 * cidx + jnp.where(sidx >= 8, 1, 0)
    jbase = (sidx & 7) * 512
    pltpu.sync_copy(idx_hbm.at[gb, pl.ds(jbase, 512)], ids_v)

    def _cv(i, c):
        v = ids_v[pl.ds(i * LANES, LANES)]
        ids_v[pl.ds(i * LANES, LANES)] = v * BATCH + gb
        return c
    lax.fori_loop(0, 512 // LANES, _cv, 0)

    def _gw(w, c):
        idxs = ids_v.at[pl.ds(w * 64, 64)]
        pltpu.async_copy(x_hbm.at[idxs], gbuf, sem).wait()
        pltpu.sync_copy(gbuf, xds_hbm.at[pl.ds(jbase + w * 64, 64), gb])
        return c
    lax.fori_loop(0, 8, _gw, 0)


def _sc_call(scores, x_flat):
    mesh = plsc.VectorSubcoreMesh(
        core_axis_name="c", subcore_axis_name="s",
        num_cores=NCORES, num_subcores=NSUB)
    return pl.kernel(
        _sc_body,
        out_type=(
            jax.ShapeDtypeStruct((BATCH, RED), jnp.int32),
            jax.ShapeDtypeStruct((BATCH, RED), jnp.float32),
            jax.ShapeDtypeStruct((RED, BATCH, DIM), jnp.float32),
        ),
        mesh=mesh,
        compiler_params=pltpu.CompilerParams(needs_layout_passes=False),
        scratch_types=[
            pltpu.VMEM((SEQ,), jnp.float32),
            pltpu.VMEM((SEQ,), jnp.int32),
            pltpu.VMEM((SEQ,), jnp.int32),
            pltpu.VMEM((SEQ,), jnp.int32),
            pltpu.VMEM((SEQ,), jnp.int32),
            pltpu.VMEM((RADIX * LANES,), jnp.int32),
            pltpu.VMEM((SEQ,), jnp.int32),
            pltpu.VMEM((RED,), jnp.float32),
            pltpu.VMEM((RED,), jnp.int32),
            pltpu.VMEM((RED,), jnp.float32),
            pltpu.VMEM((512,), jnp.int32),
            pltpu.VMEM((64, DIM), jnp.float32),
            pltpu.SemaphoreType.DMA,
        ],
    )(scores, x_flat)


def kernel(x, W):
    scores = _scores_tc(x, W.reshape(1, DIM))
    x_flat = x.reshape(SEQ * BATCH, DIM)
    indexes, weights, xds = _sc_call(scores, x_flat)
    return indexes, weights, xds
